# Initial kernel scaffold; baseline (speedup 1.0000x reference)
import jax
import jax.numpy as jnp
from jax.experimental import pallas as pl


def kernel(data_values, data_indices, idx_identity, idx_transpose, W1, b1, W2, b2, W3, b3, Wp, bp):
    raise NotImplementedError("write your pallas kernel here")


# speedup vs baseline: 4.4434x; 4.4434x over previous
import functools

import jax
import jax.numpy as jnp
from jax import lax
from jax.experimental import pallas as pl
from jax.experimental.pallas import tpu as pltpu
from jax.experimental.pallas import tpu_sc as plsc

N = 50000
NNZ = 800000
EPS = 1e-5

N_PAD = 51200
RPT = N_PAD // 16
K = 128
NCHUNK = NNZ // K
NWORK = 32
ITERS = -(-NCHUNK // NWORK)

KG = 400
NCHUNK_G = NNZ // KG
ITERS_G = -(-NCHUNK_G // NWORK)

BE = 5000
BN = 3200
BN_NODE = 1024

_MESH = plsc.VectorSubcoreMesh(core_axis_name="c", subcore_axis_name="s")
_SC_PARAMS = pltpu.CompilerParams(use_tc_tiling_on_sc=False)


def _worker_id():
    return lax.axis_index("s") * 2 + lax.axis_index("c")


@functools.partial(
    pl.kernel,
    out_type=jax.ShapeDtypeStruct((2 * 2 * N_PAD,), jnp.float32),
    mesh=_MESH,
    scratch_types=[
        pltpu.VMEM_SHARED((N_PAD,), jnp.float32),
        pltpu.VMEM_SHARED((N_PAD,), jnp.float32),
        pltpu.VMEM((K,), jnp.int32),
        pltpu.VMEM((K,), jnp.int32),
        pltpu.VMEM((K,), jnp.float32),
        pltpu.VMEM((K,), jnp.float32),
    ],
    compiler_params=_SC_PARAMS,
)
def _counts_sc(row_hbm, col_hbm, out_hbm, trow, tcol, ibr, ibc, ones, zeros):
    core = lax.axis_index("c")
    sub = lax.axis_index("s")
    w = _worker_id()
    for i in range(K // 16):
        ones[pl.ds(i * 16, 16)] = jnp.ones((16,), jnp.float32)
        zeros[pl.ds(i * 16, 16)] = jnp.zeros((16,), jnp.float32)

    @pl.loop(0, RPT // K)
    def _zero(i):
        off = sub * RPT + i * K
        pltpu.sync_copy(zeros, trow.at[pl.ds(off, K)])
        pltpu.sync_copy(zeros, tcol.at[pl.ds(off, K)])

    plsc.subcore_barrier()

    @pl.loop(0, ITERS)
    def _scatter(j):
        c = j * NWORK + w

        @pl.when(c < NCHUNK)
        def _():
            e0 = c * K
            pltpu.sync_copy(row_hbm.at[pl.ds(e0, K)], ibr)
            pltpu.sync_copy(col_hbm.at[pl.ds(e0, K)], ibc)
            pltpu.sync_copy(ones, trow.at[ibr], add=True)
            pltpu.sync_copy(ones, tcol.at[ibc], add=True)

    plsc.subcore_barrier()
    off = sub * RPT
    base = core * (2 * N_PAD)
    pltpu.sync_copy(trow.at[pl.ds(off, RPT)], out_hbm.at[pl.ds(base + off, RPT)])
    pltpu.sync_copy(tcol.at[pl.ds(off, RPT)], out_hbm.at[pl.ds(base + N_PAD + off, RPT)])


def _make_pools_sc(C, both_sides):
    G = C // 16
    S = 2 if both_sides else 1
    scratch = [
        pltpu.VMEM_SHARED((N_PAD, 16), jnp.float32),
        pltpu.VMEM_SHARED((N_PAD, 16), jnp.float32),
        pltpu.VMEM((K,), jnp.int32),
        pltpu.VMEM((K,), jnp.int32),
        pltpu.VMEM((K, 16), jnp.float32),
        pltpu.SemaphoreType.DMA,
        pltpu.SemaphoreType.DMA,
    ]

    @functools.partial(
        pl.kernel,
        out_type=jax.ShapeDtypeStruct((S * 2 * G * N_PAD, 16), jnp.float32),
        mesh=_MESH,
        scratch_types=scratch,
        compiler_params=_SC_PARAMS,
    )
    def pools(v_hbm, row_hbm, col_hbm, out_hbm, trow, tcol, ibr, ibc, vbuf,
              sem_l, sem_s):
        core = lax.axis_index("c")
        sub = lax.axis_index("s")
        w = _worker_id()

        for g in range(G):
            for r in range(K):
                vbuf[r, :] = jnp.zeros((16,), jnp.float32)

            @pl.loop(0, RPT // K)
            def _zero(i):
                off = sub * RPT + i * K
                pltpu.sync_copy(vbuf, trow.at[pl.ds(off, K), :])
                if both_sides:
                    pltpu.sync_copy(vbuf, tcol.at[pl.ds(off, K), :])

            plsc.subcore_barrier()

            @pl.loop(0, ITERS)
            def _scatter(j):
                c = j * NWORK + w

                @pl.when(c < NCHUNK)
                def _():
                    e0 = c * K
                    l0 = pltpu.async_copy(row_hbm.at[pl.ds(e0, K)], ibr, sem_l)
                    l1 = pltpu.async_copy(
                        v_hbm.at[pl.ds(e0, K), pl.ds(g * 16, 16)], vbuf, sem_l)
                    if both_sides:
                        l2 = pltpu.async_copy(col_hbm.at[pl.ds(e0, K)], ibc, sem_l)
                    l0.wait()
                    l1.wait()
                    s0 = pltpu.async_copy(vbuf, trow.at[ibr], sem_s, add=True)
                    if both_sides:
                        l2.wait()
                        s1 = pltpu.async_copy(vbuf, tcol.at[ibc], sem_s, add=True)
                    s0.wait()
                    if both_sides:
                        s1.wait()

            plsc.subcore_barrier()
            off = sub * RPT
            r0 = (core * G + g) * N_PAD + off
            pltpu.sync_copy(trow.at[pl.ds(off, RPT), :], out_hbm.at[pl.ds(r0, RPT), :])
            if both_sides:
                r1 = ((2 + core) * G + g) * N_PAD + off
                pltpu.sync_copy(tcol.at[pl.ds(off, RPT), :], out_hbm.at[pl.ds(r1, RPT), :])
            plsc.subcore_barrier()

    return pools


def _make_gathers_sc(C, D):
    @functools.partial(
        pl.kernel,
        out_type=[
            jax.ShapeDtypeStruct((NNZ, C), jnp.float32),
            jax.ShapeDtypeStruct((NNZ, C), jnp.float32),
            jax.ShapeDtypeStruct((NNZ, D), jnp.float32),
            jax.ShapeDtypeStruct((NNZ, D), jnp.float32),
        ],
        mesh=_MESH,
        scratch_types=[
            pltpu.VMEM((KG,), jnp.int32),
            pltpu.VMEM((KG,), jnp.int32),
            pltpu.VMEM((KG,), jnp.int32),
            pltpu.VMEM((KG,), jnp.int32),
            pltpu.VMEM((KG, C), jnp.float32),
            pltpu.VMEM((KG, C), jnp.float32),
            pltpu.VMEM((KG, D), jnp.float32),
            pltpu.VMEM((KG, D), jnp.float32),
            pltpu.SemaphoreType.DMA,
            pltpu.SemaphoreType.DMA,
            pltpu.SemaphoreType.DMA,
        ],
        compiler_params=_SC_PARAMS,
    )
    def gathers(v_hbm, it_hbm, ii_hbm, a_hbm, b_hbm, row_hbm, col_hbm,
                t_out, i_out, pa_out, pb_out,
                ibt, ibi, ibr, ibc, buft, bufi, bufa, bufb,
                sem_i, sem_g, sem_w):
        w = _worker_id()

        @pl.loop(0, ITERS_G)
        def _gather(j):
            c = j * NWORK + w

            @pl.when(c < NCHUNK_G)
            def _():
                e0 = c * KG
                d0 = pltpu.async_copy(it_hbm.at[pl.ds(e0, KG)], ibt, sem_i)
                d1 = pltpu.async_copy(ii_hbm.at[pl.ds(e0, KG)], ibi, sem_i)
                d2 = pltpu.async_copy(row_hbm.at[pl.ds(e0, KG)], ibr, sem_i)
                d3 = pltpu.async_copy(col_hbm.at[pl.ds(e0, KG)], ibc, sem_i)
                d0.wait()
                d1.wait()
                d2.wait()
                d3.wait()
                g0 = pltpu.async_copy(v_hbm.at[ibt], buft, sem_g)
                g1 = pltpu.async_copy(v_hbm.at[ibi], bufi, sem_g)
                g2 = pltpu.async_copy(a_hbm.at[ibr], bufa, sem_g)
                g3 = pltpu.async_copy(b_hbm.at[ibc], bufb, sem_g)
                g0.wait()
                g1.wait()
                g2.wait()
                g3.wait()
                w0 = pltpu.async_copy(buft, t_out.at[pl.ds(e0, KG), :], sem_w)
                w1 = pltpu.async_copy(bufi, i_out.at[pl.ds(e0, KG), :], sem_w)
                w2 = pltpu.async_copy(bufa, pa_out.at[pl.ds(e0, KG), :], sem_w)
                w3 = pltpu.async_copy(bufb, pb_out.at[pl.ds(e0, KG), :], sem_w)
                w0.wait()
                w1.wait()
                w2.wait()
                w3.wait()

    return gathers


def _make_node_tc(C, D):
    G = C // 16

    def body(prow, pcol, cnts, w3, w4, a_out, b_out, gsum_out):
        sr = prow[0, 0] + prow[0, 1]
        sc = pcol[0, 0] + pcol[0, 1]
        srow = jnp.concatenate([sr[g] for g in range(G)], axis=-1)
        scol = jnp.concatenate([sc[g] for g in range(G)], axis=-1)
        cr = cnts[0, 0] + cnts[1, 0]
        cc = cnts[0, 1] + cnts[1, 1]
        rp = srow / jnp.maximum(cr, 1.0)[:, None]
        cp = scol / jnp.maximum(cc, 1.0)[:, None]
        a_out[...] = jnp.dot(rp, w3[...], preferred_element_type=jnp.float32)
        b_out[...] = jnp.dot(cp, w4[...], preferred_element_type=jnp.float32)
        gs = jnp.sum(srow, axis=0, keepdims=True)
        pid = pl.program_id(0)

        @pl.when(pid == 0)
        def _():
            gsum_out[...] = gs

        @pl.when(pid != 0)
        def _():
            gsum_out[...] += gs

    return pl.pallas_call(
        body,
        grid=(N_PAD // BN_NODE,),
        in_specs=[
            pl.BlockSpec((1, 2, G, BN_NODE, 16), lambda n: (0, 0, 0, n, 0)),
            pl.BlockSpec((1, 2, G, BN_NODE, 16), lambda n: (1, 0, 0, n, 0)),
            pl.BlockSpec((2, 2, BN_NODE), lambda n: (0, 0, n)),
            pl.BlockSpec((C, D), lambda n: (0, 0)),
            pl.BlockSpec((C, D), lambda n: (0, 0)),
        ],
        out_specs=[
            pl.BlockSpec((BN_NODE, D), lambda n: (n, 0)),
            pl.BlockSpec((BN_NODE, D), lambda n: (n, 0)),
            pl.BlockSpec((1, C), lambda n: (0, 0)),
        ],
        out_shape=[
            jax.ShapeDtypeStruct((N_PAD, D), jnp.float32),
            jax.ShapeDtypeStruct((N_PAD, D), jnp.float32),
            jax.ShapeDtypeStruct((1, C), jnp.float32),
        ],
    )


def _make_edge_tc(C, D):
    def body(v, t, i_, pa, pb, w0, w1, w2, w5, b, gsum, out, stats):
        beff = b[...] + jnp.dot(gsum[...] * (1.0 / NNZ), w5[...],
                                preferred_element_type=jnp.float32)
        acc = (jnp.dot(v[...], w0[...], preferred_element_type=jnp.float32)
               + jnp.dot(t[...], w1[...], preferred_element_type=jnp.float32)
               + jnp.dot(i_[...], w2[...], preferred_element_type=jnp.float32)
               + pa[...] + pb[...] + beff)
        out[...] = acc
        st = jnp.concatenate([jnp.sum(acc, axis=0, keepdims=True),
                              jnp.sum(acc * acc, axis=0, keepdims=True)], axis=0)
        pid = pl.program_id(0)

        @pl.when(pid == 0)
        def _():
            stats[...] = st

        @pl.when(pid != 0)
        def _():
            stats[...] += st

    return pl.pallas_call(
        body,
        grid=(NNZ // BE,),
        in_specs=[
            pl.BlockSpec((BE, C), lambda e: (e, 0)),
            pl.BlockSpec((BE, C), lambda e: (e, 0)),
            pl.BlockSpec((BE, C), lambda e: (e, 0)),
            pl.BlockSpec((BE, D), lambda e: (e, 0)),
            pl.BlockSpec((BE, D), lambda e: (e, 0)),
            pl.BlockSpec((C, D), lambda e: (0, 0)),
            pl.BlockSpec((C, D), lambda e: (0, 0)),
            pl.BlockSpec((C, D), lambda e: (0, 0)),
            pl.BlockSpec((C, D), lambda e: (0, 0)),
            pl.BlockSpec((1, D), lambda e: (0, 0)),
            pl.BlockSpec((1, C), lambda e: (0, 0)),
        ],
        out_specs=[
            pl.BlockSpec((BE, D), lambda e: (e, 0)),
            pl.BlockSpec((2, D), lambda e: (0, 0)),
        ],
        out_shape=[
            jax.ShapeDtypeStruct((NNZ, D), jnp.float32),
            jax.ShapeDtypeStruct((2, D), jnp.float32),
        ],
    )


def _make_norm_tc(D):
    def body(x, stats, out):
        m = stats[0:1, :] * (1.0 / NNZ)
        var = stats[1:2, :] * (1.0 / NNZ) - m * m
        inv = lax.rsqrt(var + EPS)
        out[...] = jnp.maximum((x[...] - m) * inv, 0.0)

    return pl.pallas_call(
        body,
        grid=(NNZ // BE,),
        in_specs=[
            pl.BlockSpec((BE, D), lambda e: (e, 0)),
            pl.BlockSpec((2, D), lambda e: (0, 0)),
        ],
        out_specs=pl.BlockSpec((BE, D), lambda e: (e, 0)),
        out_shape=jax.ShapeDtypeStruct((NNZ, D), jnp.float32),
    )


def _make_final_tc(C):
    G = C // 16

    def body(prow, cnts, wp, bp, out):
        sr = prow[0, 0] + prow[0, 1]
        srow = jnp.concatenate([sr[g] for g in range(G)], axis=-1)
        cr = cnts[0, 0] + cnts[1, 0]
        ent = srow / jnp.maximum(cr, 1.0)[:, None]
        out[...] = jnp.dot(ent, wp[...], preferred_element_type=jnp.float32) + bp[...]

    return pl.pallas_call(
        body,
        grid=(N_PAD // BN,),
        in_specs=[
            pl.BlockSpec((1, 2, G, BN, 16), lambda n: (0, 0, 0, n, 0)),
            pl.BlockSpec((2, 2, BN), lambda n: (0, 0, n)),
            pl.BlockSpec((C, 1), lambda n: (0, 0)),
            pl.BlockSpec((1, 1), lambda n: (0, 0)),
        ],
        out_specs=pl.BlockSpec((BN, 1), lambda n: (n, 0)),
        out_shape=jax.ShapeDtypeStruct((N_PAD, 1), jnp.float32),
    )


def kernel(data_values, data_indices, idx_identity, idx_transpose,
           W1, b1, W2, b2, W3, b3, Wp, bp):
    row = data_indices[0].astype(jnp.int32)
    col = data_indices[1].astype(jnp.int32)
    it = idx_transpose.astype(jnp.int32)
    ii = idx_identity.astype(jnp.int32)

    cnts = _counts_sc(row, col).reshape(2, 2, N_PAD)

    v = data_values
    for (W, b), (C, D) in zip(((W1, b1), (W2, b2), (W3, b3)),
                              ((16, 32), (32, 64), (64, 32))):
        G = C // 16
        p5 = _make_pools_sc(C, True)(v, row, col).reshape(2, 2, G, N_PAD, 16)
        A, B, gsum = _make_node_tc(C, D)(p5, p5, cnts, W[3], W[4])
        t, i_, pa, pb = _make_gathers_sc(C, D)(v, it, ii, A, B, row, col)
        raw, stats = _make_edge_tc(C, D)(v, t, i_, pa, pb, W[0], W[1], W[2],
                                         W[5], b.reshape(1, D), gsum)
        v = _make_norm_tc(D)(raw, stats)

    pF = _make_pools_sc(32, False)(v, row, col).reshape(1, 2, 2, N_PAD, 16)
    out = _make_final_tc(32)(pF, cnts, Wp, bp.reshape(1, 1))
    return out[:N]


# speedup vs baseline: 4.9910x; 1.1232x over previous
import functools

import jax
import jax.numpy as jnp
from jax import lax
from jax.experimental import pallas as pl
from jax.experimental.pallas import tpu as pltpu
from jax.experimental.pallas import tpu_sc as plsc

N = 50000
NNZ = 800000
EPS = 1e-5

N_PAD = 51200
RPT = N_PAD // 16
K = 128
NCHUNK = NNZ // K
NWORK = 32
ITERS = -(-NCHUNK // NWORK)

KG = 256
NCHUNK_G = NNZ // KG
ITERS_G = -(-NCHUNK_G // NWORK)

BE = 5000
BN = 3200
BN_NODE = 1024

_MESH = plsc.VectorSubcoreMesh(core_axis_name="c", subcore_axis_name="s")
_SC_PARAMS = pltpu.CompilerParams(use_tc_tiling_on_sc=False)


def _worker_id():
    return lax.axis_index("s") * 2 + lax.axis_index("c")


def _make_pools_sc(C, both_sides, with_counts=False, apply_norm=False):
    G = C // 16
    S = 2 if both_sides else 1
    out_type = [jax.ShapeDtypeStruct((S * 2 * G * N_PAD, 16), jnp.float32)]
    scratch = [
        pltpu.VMEM_SHARED((N_PAD, 16), jnp.float32),
        pltpu.VMEM_SHARED((N_PAD, 16), jnp.float32),
        pltpu.VMEM((2, K), jnp.int32),
        pltpu.VMEM((2, K), jnp.int32),
        pltpu.VMEM((2, K, 16), jnp.float32),
        pltpu.SemaphoreType.DMA,
        pltpu.SemaphoreType.DMA,
    ]
    if with_counts:
        out_type.append(jax.ShapeDtypeStruct((2 * 2 * N_PAD,), jnp.float32))
        scratch += [
            pltpu.VMEM_SHARED((N_PAD,), jnp.float32),
            pltpu.VMEM_SHARED((N_PAD,), jnp.float32),
            pltpu.VMEM((K,), jnp.float32),
            pltpu.VMEM((K,), jnp.float32),
        ]
    if apply_norm:
        scratch.append(pltpu.VMEM((2 * C,), jnp.float32))

    @functools.partial(
        pl.kernel,
        out_type=out_type,
        mesh=_MESH,
        scratch_types=scratch,
        compiler_params=_SC_PARAMS,
    )
    def pools(v_hbm, *args):
        if apply_norm:
            normp_hbm, row_hbm, col_hbm, out_hbm = args[:4]
            trow, tcol, ibr, ibc, vbuf, sem_l, sem_s, npv = args[4:]
        elif with_counts:
            row_hbm, col_hbm, out_hbm, cnt_hbm = args[:4]
            (trow, tcol, ibr, ibc, vbuf, sem_l, sem_s,
             tcr, tcc, ones, zc) = args[4:]
        else:
            row_hbm, col_hbm, out_hbm = args[:3]
            trow, tcol, ibr, ibc, vbuf, sem_l, sem_s = args[3:]
        if apply_norm:
            pltpu.sync_copy(normp_hbm, npv)
        core = lax.axis_index("c")
        sub = lax.axis_index("s")
        w = _worker_id()

        def c_of(j):
            return j * NWORK + w

        for g in range(G):
            for r in range(K):
                vbuf[0, r, :] = jnp.zeros((16,), jnp.float32)
            if with_counts and g == 0:
                for i2 in range(K // 16):
                    ones[pl.ds(i2 * 16, 16)] = jnp.ones((16,), jnp.float32)
                    zc[pl.ds(i2 * 16, 16)] = jnp.zeros((16,), jnp.float32)

            @pl.loop(0, RPT // K)
            def _zero(i):
                off = sub * RPT + i * K
                pltpu.sync_copy(vbuf.at[0], trow.at[pl.ds(off, K), :])
                if both_sides:
                    pltpu.sync_copy(vbuf.at[0], tcol.at[pl.ds(off, K), :])
                if with_counts and g == 0:
                    pltpu.sync_copy(zc, tcr.at[pl.ds(off, K)])
                    pltpu.sync_copy(zc, tcc.at[pl.ds(off, K)])

            plsc.subcore_barrier()

            def issue_loads(jj, p):
                cc = c_of(jj)

                @pl.when(cc < NCHUNK)
                def _():
                    e0 = cc * K
                    pltpu.async_copy(row_hbm.at[pl.ds(e0, K)], ibr.at[p], sem_l)
                    if both_sides or (with_counts and g == 0):
                        pltpu.async_copy(col_hbm.at[pl.ds(e0, K)], ibc.at[p], sem_l)
                    pltpu.async_copy(
                        v_hbm.at[pl.ds(e0, K), pl.ds(g * 16, 16)], vbuf.at[p], sem_l)

            def wait_loads(jj, p):
                cc = c_of(jj)

                @pl.when(cc < NCHUNK)
                def _():
                    e0 = cc * K
                    pltpu.make_async_copy(
                        row_hbm.at[pl.ds(e0, K)], ibr.at[p], sem_l).wait()
                    if both_sides or (with_counts and g == 0):
                        pltpu.make_async_copy(
                            col_hbm.at[pl.ds(e0, K)], ibc.at[p], sem_l).wait()
                    pltpu.make_async_copy(
                        v_hbm.at[pl.ds(e0, K), pl.ds(g * 16, 16)],
                        vbuf.at[p], sem_l).wait()

            def process(jj, p):
                cc = c_of(jj)

                @pl.when(cc < NCHUNK)
                def _():
                    if apply_norm:
                        mvec = npv[pl.ds(g * 16, 16)]
                        ivec = npv[pl.ds(C + g * 16, 16)]
                        for r in range(K):
                            vbuf[p, r, :] = jnp.maximum(
                                (vbuf[p, r, :] - mvec) * ivec, 0.0)
                    ss = [pltpu.async_copy(
                        vbuf.at[p], trow.at[ibr.at[p]], sem_s, add=True)]
                    if both_sides:
                        ss.append(pltpu.async_copy(
                            vbuf.at[p], tcol.at[ibc.at[p]], sem_s, add=True))
                    if with_counts and g == 0:
                        ss.append(pltpu.async_copy(
                            ones, tcr.at[ibr.at[p]], sem_s, add=True))
                        ss.append(pltpu.async_copy(
                            ones, tcc.at[ibc.at[p]], sem_s, add=True))
                    for s in ss:
                        s.wait()

            issue_loads(0, 0)

            @pl.loop(0, ITERS, step=2)
            def _main(j):
                wait_loads(j, 0)
                issue_loads(j + 1, 1)
                process(j, 0)
                wait_loads(j + 1, 1)
                issue_loads(j + 2, 0)
                process(j + 1, 1)

            plsc.subcore_barrier()
            off = sub * RPT
            r0 = (core * G + g) * N_PAD + off
            pltpu.sync_copy(trow.at[pl.ds(off, RPT), :], out_hbm.at[pl.ds(r0, RPT), :])
            if both_sides:
                r1 = ((2 + core) * G + g) * N_PAD + off
                pltpu.sync_copy(tcol.at[pl.ds(off, RPT), :], out_hbm.at[pl.ds(r1, RPT), :])
            if with_counts and g == 0:
                base = core * (2 * N_PAD)
                pltpu.sync_copy(tcr.at[pl.ds(off, RPT)],
                                cnt_hbm.at[pl.ds(base + off, RPT)])
                pltpu.sync_copy(tcc.at[pl.ds(off, RPT)],
                                cnt_hbm.at[pl.ds(base + N_PAD + off, RPT)])
            plsc.subcore_barrier()

    return pools


def _make_gathers_sc(C, D):
    @functools.partial(
        pl.kernel,
        out_type=[
            jax.ShapeDtypeStruct((NNZ, C), jnp.float32),
            jax.ShapeDtypeStruct((NNZ, C), jnp.float32),
            jax.ShapeDtypeStruct((NNZ, D), jnp.float32),
            jax.ShapeDtypeStruct((NNZ, D), jnp.float32),
        ],
        mesh=_MESH,
        scratch_types=[
            pltpu.VMEM((2, KG), jnp.int32),
            pltpu.VMEM((2, KG), jnp.int32),
            pltpu.VMEM((2, KG), jnp.int32),
            pltpu.VMEM((2, KG), jnp.int32),
            pltpu.VMEM((2, KG, C), jnp.float32),
            pltpu.VMEM((2, KG, C), jnp.float32),
            pltpu.VMEM((2, KG, D), jnp.float32),
            pltpu.VMEM((2, KG, D), jnp.float32),
            pltpu.SemaphoreType.DMA,
            pltpu.SemaphoreType.DMA,
            pltpu.SemaphoreType.DMA,
        ],
        compiler_params=_SC_PARAMS,
    )
    def gathers(v_hbm, it_hbm, ii_hbm, a_hbm, b_hbm, row_hbm, col_hbm,
                t_out, i_out, pa_out, pb_out,
                ibt, ibi, ibr, ibc, buft, bufi, bufa, bufb,
                sem_i, sem_g, sem_w):
        w = _worker_id()

        def c_of(jj):
            return jj * NWORK + w

        def issue_loads(jj, p):
            cc = c_of(jj)

            @pl.when(cc < NCHUNK_G)
            def _():
                e0 = cc * KG
                pltpu.async_copy(it_hbm.at[pl.ds(e0, KG)], ibt.at[p], sem_i)
                pltpu.async_copy(ii_hbm.at[pl.ds(e0, KG)], ibi.at[p], sem_i)
                pltpu.async_copy(row_hbm.at[pl.ds(e0, KG)], ibr.at[p], sem_i)
                pltpu.async_copy(col_hbm.at[pl.ds(e0, KG)], ibc.at[p], sem_i)

        def wait_loads(jj, p):
            cc = c_of(jj)

            @pl.when(cc < NCHUNK_G)
            def _():
                e0 = cc * KG
                pltpu.make_async_copy(it_hbm.at[pl.ds(e0, KG)], ibt.at[p], sem_i).wait()
                pltpu.make_async_copy(ii_hbm.at[pl.ds(e0, KG)], ibi.at[p], sem_i).wait()
                pltpu.make_async_copy(row_hbm.at[pl.ds(e0, KG)], ibr.at[p], sem_i).wait()
                pltpu.make_async_copy(col_hbm.at[pl.ds(e0, KG)], ibc.at[p], sem_i).wait()

        def drain_wb(jj, p):
            cc = c_of(jj)

            @pl.when((jj >= 0) & (cc < NCHUNK_G))
            def _():
                e0 = cc * KG
                pltpu.make_async_copy(buft.at[p], t_out.at[pl.ds(e0, KG), :], sem_w).wait()
                pltpu.make_async_copy(bufi.at[p], i_out.at[pl.ds(e0, KG), :], sem_w).wait()
                pltpu.make_async_copy(bufa.at[p], pa_out.at[pl.ds(e0, KG), :], sem_w).wait()
                pltpu.make_async_copy(bufb.at[p], pb_out.at[pl.ds(e0, KG), :], sem_w).wait()

        def process(jj, p):
            cc = c_of(jj)

            @pl.when(cc < NCHUNK_G)
            def _():
                e0 = cc * KG
                g0 = pltpu.async_copy(v_hbm.at[ibt.at[p]], buft.at[p], sem_g)
                g1 = pltpu.async_copy(v_hbm.at[ibi.at[p]], bufi.at[p], sem_g)
                g2 = pltpu.async_copy(a_hbm.at[ibr.at[p]], bufa.at[p], sem_g)
                g3 = pltpu.async_copy(b_hbm.at[ibc.at[p]], bufb.at[p], sem_g)
                g0.wait()
                g1.wait()
                g2.wait()
                g3.wait()
                pltpu.async_copy(buft.at[p], t_out.at[pl.ds(e0, KG), :], sem_w)
                pltpu.async_copy(bufi.at[p], i_out.at[pl.ds(e0, KG), :], sem_w)
                pltpu.async_copy(bufa.at[p], pa_out.at[pl.ds(e0, KG), :], sem_w)
                pltpu.async_copy(bufb.at[p], pb_out.at[pl.ds(e0, KG), :], sem_w)

        issue_loads(0, 0)

        @pl.loop(0, ITERS_G, step=2)
        def _main(j):
            wait_loads(j, 0)
            issue_loads(j + 1, 1)
            drain_wb(j - 2, 0)
            process(j, 0)
            wait_loads(j + 1, 1)
            issue_loads(j + 2, 0)
            drain_wb(j - 1, 1)
            process(j + 1, 1)

        drain_wb(ITERS_G - 2, 0)
        drain_wb(ITERS_G - 1, 1)

    return gathers


def _make_node_tc(C, D):
    G = C // 16

    def body(prow, pcol, cnts, w3, w4, a_out, b_out, gsum_out):
        sr = prow[0, 0] + prow[0, 1]
        sc = pcol[0, 0] + pcol[0, 1]
        srow = jnp.concatenate([sr[g] for g in range(G)], axis=-1)
        scol = jnp.concatenate([sc[g] for g in range(G)], axis=-1)
        cr = cnts[0, 0] + cnts[1, 0]
        cc = cnts[0, 1] + cnts[1, 1]
        rp = srow / jnp.maximum(cr, 1.0)[:, None]
        cp = scol / jnp.maximum(cc, 1.0)[:, None]
        a_out[...] = jnp.dot(rp, w3[...], preferred_element_type=jnp.float32)
        b_out[...] = jnp.dot(cp, w4[...], preferred_element_type=jnp.float32)
        gs = jnp.sum(srow, axis=0, keepdims=True)
        pid = pl.program_id(0)

        @pl.when(pid == 0)
        def _():
            gsum_out[...] = gs

        @pl.when(pid != 0)
        def _():
            gsum_out[...] += gs

    return pl.pallas_call(
        body,
        grid=(N_PAD // BN_NODE,),
        in_specs=[
            pl.BlockSpec((1, 2, G, BN_NODE, 16), lambda n: (0, 0, 0, n, 0)),
            pl.BlockSpec((1, 2, G, BN_NODE, 16), lambda n: (1, 0, 0, n, 0)),
            pl.BlockSpec((2, 2, BN_NODE), lambda n: (0, 0, n)),
            pl.BlockSpec((C, D), lambda n: (0, 0)),
            pl.BlockSpec((C, D), lambda n: (0, 0)),
        ],
        out_specs=[
            pl.BlockSpec((BN_NODE, D), lambda n: (n, 0)),
            pl.BlockSpec((BN_NODE, D), lambda n: (n, 0)),
            pl.BlockSpec((1, C), lambda n: (0, 0)),
        ],
        out_shape=[
            jax.ShapeDtypeStruct((N_PAD, D), jnp.float32),
            jax.ShapeDtypeStruct((N_PAD, D), jnp.float32),
            jax.ShapeDtypeStruct((1, C), jnp.float32),
        ],
    )


def _make_edge_tc(C, D, in_norm):
    def body(*args):
        if in_norm:
            (v, t, i_, pa, pb, w0, w1, w2, w5, b, gsum, np_in, out, stats) = args
            m0 = np_in[0:1, :]
            iv0 = np_in[1:2, :]
            vv = jnp.maximum((v[...] - m0) * iv0, 0.0)
            tt = jnp.maximum((t[...] - m0) * iv0, 0.0)
            ii2 = jnp.maximum((i_[...] - m0) * iv0, 0.0)
        else:
            (v, t, i_, pa, pb, w0, w1, w2, w5, b, gsum, out, stats) = args
            vv, tt, ii2 = v[...], t[...], i_[...]
        beff = b[...] + jnp.dot(gsum[...] * (1.0 / NNZ), w5[...],
                                preferred_element_type=jnp.float32)
        acc = (jnp.dot(vv, w0[...], preferred_element_type=jnp.float32)
               + jnp.dot(tt, w1[...], preferred_element_type=jnp.float32)
               + jnp.dot(ii2, w2[...], preferred_element_type=jnp.float32)
               + pa[...] + pb[...] + beff)
        out[...] = acc
        st = jnp.concatenate([jnp.sum(acc, axis=0, keepdims=True),
                              jnp.sum(acc * acc, axis=0, keepdims=True)], axis=0)
        pid = pl.program_id(0)

        @pl.when(pid == 0)
        def _():
            stats[...] = st

        @pl.when(pid != 0)
        def _():
            stats[...] += st

        @pl.when(pid == NNZ // BE - 1)
        def _():
            s = stats[...]
            m = s[0:1, :] * (1.0 / NNZ)
            var = s[1:2, :] * (1.0 / NNZ) - m * m
            stats[...] = jnp.concatenate([m, lax.rsqrt(var + EPS)], axis=0)

    in_specs = [
        pl.BlockSpec((BE, C), lambda e: (e, 0)),
        pl.BlockSpec((BE, C), lambda e: (e, 0)),
        pl.BlockSpec((BE, C), lambda e: (e, 0)),
        pl.BlockSpec((BE, D), lambda e: (e, 0)),
        pl.BlockSpec((BE, D), lambda e: (e, 0)),
        pl.BlockSpec((C, D), lambda e: (0, 0)),
        pl.BlockSpec((C, D), lambda e: (0, 0)),
        pl.BlockSpec((C, D), lambda e: (0, 0)),
        pl.BlockSpec((C, D), lambda e: (0, 0)),
        pl.BlockSpec((1, D), lambda e: (0, 0)),
        pl.BlockSpec((1, C), lambda e: (0, 0)),
    ]
    if in_norm:
        in_specs.append(pl.BlockSpec((2, C), lambda e: (0, 0)))
    return pl.pallas_call(
        body,
        grid=(NNZ // BE,),
        in_specs=in_specs,
        out_specs=[
            pl.BlockSpec((BE, D), lambda e: (e, 0)),
            pl.BlockSpec((2, D), lambda e: (0, 0)),
        ],
        out_shape=[
            jax.ShapeDtypeStruct((NNZ, D), jnp.float32),
            jax.ShapeDtypeStruct((2, D), jnp.float32),
        ],
    )


def _make_final_tc(C):
    G = C // 16

    def body(prow, cnts, wp, bp, out):
        sr = prow[0, 0] + prow[0, 1]
        srow = jnp.concatenate([sr[g] for g in range(G)], axis=-1)
        cr = cnts[0, 0] + cnts[1, 0]
        ent = srow / jnp.maximum(cr, 1.0)[:, None]
        out[...] = jnp.dot(ent, wp[...], preferred_element_type=jnp.float32) + bp[...]

    return pl.pallas_call(
        body,
        grid=(N_PAD // BN,),
        in_specs=[
            pl.BlockSpec((1, 2, G, BN, 16), lambda n: (0, 0, 0, n, 0)),
            pl.BlockSpec((2, 2, BN), lambda n: (0, 0, n)),
            pl.BlockSpec((C, 1), lambda n: (0, 0)),
            pl.BlockSpec((1, 1), lambda n: (0, 0)),
        ],
        out_specs=pl.BlockSpec((BN, 1), lambda n: (n, 0)),
        out_shape=jax.ShapeDtypeStruct((N_PAD, 1), jnp.float32),
    )


def kernel(data_values, data_indices, idx_identity, idx_transpose,
           W1, b1, W2, b2, W3, b3, Wp, bp):
    row = data_indices[0].astype(jnp.int32)
    col = data_indices[1].astype(jnp.int32)
    it = idx_transpose.astype(jnp.int32)
    ii = idx_identity.astype(jnp.int32)

    v = data_values
    cnts = None
    normp = None
    for (W, b), (C, D) in zip(((W1, b1), (W2, b2), (W3, b3)),
                              ((16, 32), (32, 64), (64, 32))):
        G = C // 16
        if cnts is None:
            p5, cnts_flat = _make_pools_sc(C, True, with_counts=True)(v, row, col)
            cnts = cnts_flat.reshape(2, 2, N_PAD)
        else:
            (p5,) = _make_pools_sc(C, True, apply_norm=True)(
                v, normp.reshape(2 * C), row, col)
        p5 = p5.reshape(2, 2, G, N_PAD, 16)
        A, B, gsum = _make_node_tc(C, D)(p5, p5, cnts, W[3], W[4])
        t, i_, pa, pb = _make_gathers_sc(C, D)(v, it, ii, A, B, row, col)
        if normp is None:
            raw, normp = _make_edge_tc(C, D, False)(
                v, t, i_, pa, pb, W[0], W[1], W[2], W[5], b.reshape(1, D), gsum)
        else:
            raw, normp = _make_edge_tc(C, D, True)(
                v, t, i_, pa, pb, W[0], W[1], W[2], W[5], b.reshape(1, D),
                gsum, normp)
        v = raw

    (pF,) = _make_pools_sc(32, False, apply_norm=True)(
        v, normp.reshape(2 * 32), row, col)
    pF = pF.reshape(1, 2, 2, N_PAD, 16)
    out = _make_final_tc(32)(pF, cnts, Wp, bp.reshape(1, 1))
    return out[:N]
